# baseline (speedup 1.0000x reference)
import functools

import jax
import jax.numpy as jnp
from jax import lax
from jax.experimental import pallas as pl
from jax.experimental.pallas import tpu as pltpu
from jax.experimental.pallas import tpu_sc as plsc

K = 16
EPS = 1e-5
NA = 256
NB = 128
DPAD = 16
DROW = 256

SC_NC = 2
SC_NS = 16
SC_NW = SC_NC * SC_NS
GCHUNK = 256
NCHUNK = 4


def _ce(v, i, a, b):
    le = v[a] <= v[b]
    va, vb = jnp.where(le, v[a], v[b]), jnp.where(le, v[b], v[a])
    ia, ib = jnp.where(le, i[a], i[b]), jnp.where(le, i[b], i[a])
    v[a], v[b], i[a], i[b] = va, vb, ia, ib


def _knn_body(q_ref, p_ref, idx_ref):
    q = q_ref[...]
    p = p_ref[0]
    nt = p.shape[1]
    w = nt // K
    psq = jnp.sum(p * p, axis=0, keepdims=True)
    d = psq - 2.0 * jnp.dot(q, p, preferred_element_type=jnp.float32)
    iota = lax.broadcasted_iota(jnp.int32, (d.shape[0], w), 1)
    v = [d[:, c * w:(c + 1) * w] for c in range(K)]
    i = [iota + c * w for c in range(K)]

    for k in (2, 4, 8, 16):
        j = k // 2
        while j >= 1:
            for a in range(K):
                b = a ^ j
                if b > a:
                    if (a & k) == 0:
                        _ce(v, i, a, b)
                    else:
                        _ce(v, i, b, a)
            j //= 2

    while w > 32:
        h = w // 2
        lo_v = [x[:, :h] for x in v]
        hi_v = [x[:, h:] for x in v]
        lo_i = [x[:, :h] for x in i]
        hi_i = [x[:, h:] for x in i]
        for a in range(K):
            le = lo_v[a] <= hi_v[K - 1 - a]
            v[a] = jnp.where(le, lo_v[a], hi_v[K - 1 - a])
            i[a] = jnp.where(le, lo_i[a], hi_i[K - 1 - a])
        w = h
        if w > 32:
            for j in (8, 4, 2, 1):
                for a in range(K):
                    b = a ^ j
                    if b > a:
                        _ce(v, i, a, b)

    dv = jnp.concatenate(v, axis=1)
    di = jnp.concatenate(i, axis=1)
    col = jax.lax.broadcasted_iota(jnp.int32, (dv.shape[0], K), 1)
    acc = jnp.zeros((dv.shape[0], K), jnp.int32)
    big = jnp.float32(3e38)
    for k in range(K):
        rmin = jnp.min(dv, axis=1, keepdims=True)
        eqm = dv == rmin
        cand = jnp.where(eqm, di, jnp.int32(1 << 30))
        sel = jnp.min(cand, axis=1, keepdims=True)
        dv = jnp.where(eqm & (di == sel), big, dv)
        acc = jnp.where(col == k, sel, acc)
    idx_ref[...] = acc


def _knn_call(qpos, fpos, b, n, nt):
    grid = (b, n // NA)
    return pl.pallas_call(
        _knn_body,
        grid=grid,
        in_specs=[
            pl.BlockSpec((NA, 8), lambda i, j: (i * (n // NA) + j, 0)),
            pl.BlockSpec((1, 8, nt), lambda i, j: (i, 0, 0)),
        ],
        out_specs=pl.BlockSpec((NA, K), lambda i, j: (i * (n // NA) + j, 0)),
        out_shape=jax.ShapeDtypeStruct((b * n, K), jnp.int32),
    )(qpos, fpos)


def _gather_body(table_hbm, idx_hbm, out_hbm, idx_v, rows_v, sem):
    wid = lax.axis_index("s") * SC_NC + lax.axis_index("c")
    rows_total = out_hbm.shape[0]
    per_w = rows_total // SC_NW
    base = wid * per_w

    def body(j, carry):
        off = base + j * GCHUNK
        pltpu.sync_copy(idx_hbm.at[pl.ds(off, GCHUNK)], idx_v)
        pltpu.async_copy(table_hbm.at[idx_v], rows_v, sem).wait()
        pltpu.sync_copy(rows_v, out_hbm.at[pl.ds(off, GCHUNK)])
        return carry

    lax.fori_loop(0, per_w // GCHUNK, body, 0)


def _gather_call(table, idx_flat):
    rows = idx_flat.shape[0]
    mesh = plsc.VectorSubcoreMesh(core_axis_name="c", subcore_axis_name="s")
    f = functools.partial(
        pl.kernel,
        mesh=mesh,
        out_type=jax.ShapeDtypeStruct((rows, DROW), jnp.float32),
        scratch_types=[
            pltpu.VMEM((GCHUNK,), jnp.int32),
            pltpu.VMEM((GCHUNK, DROW), jnp.float32),
            pltpu.SemaphoreType.DMA,
        ],
    )(_gather_body)
    return f(table, idx_flat)


def _mlp_body(g_ref, kf_ref, kp_ref, pw1_ref, pb1_ref, pw2_ref, pb2_ref,
              aw1_ref, ab1_ref, aw2_ref, ab2_ref, out_ref):
    g = g_ref[...].reshape(K * NB, DROW)
    gi = lax.bitcast_convert_type(g[:, :128], jnp.int32)
    f_even = lax.bitcast_convert_type(gi << 16, jnp.float32)
    f_odd = lax.bitcast_convert_type(
        gi & jnp.int32(-65536), jnp.float32)
    gfeat = jnp.concatenate((f_even, f_odd), axis=1)
    gpos = g[:, 128:128 + DPAD]
    kf = kf_ref[...]
    kp = kp_ref[...]
    kfb = (jnp.broadcast_to(kf[None], (K, NB, 256)) - gfeat.reshape(K, NB, 256)
           ).reshape(K * NB, 256)
    kpb = (jnp.broadcast_to(kp[None], (K, NB, DPAD)) - gpos.reshape(K, NB, DPAD)
           ).reshape(K * NB, DPAD)

    h = jnp.maximum(
        jnp.dot(kpb, pw1_ref[...], preferred_element_type=jnp.float32)
        + pb1_ref[...], 0.0)
    pe = jnp.dot(h, pw2_ref[...], preferred_element_type=jnp.float32) + pb2_ref[...]

    x = (kfb + pe).astype(jnp.bfloat16)
    a = jnp.maximum(
        jnp.dot(x, aw1_ref[...],
                preferred_element_type=jnp.float32).astype(jnp.bfloat16)
        + ab1_ref[...], jnp.bfloat16(0.0))
    logits = (jnp.dot(a, aw2_ref[...], preferred_element_type=jnp.float32)
              + ab2_ref[...])

    l3 = logits.reshape(K, NB, 256)
    v3 = (gfeat + pe).reshape(K, NB, 256)
    m = l3[0]
    for k in range(1, K):
        m = jnp.maximum(m, l3[k])
    num = jnp.zeros((NB, 256), jnp.float32)
    den = jnp.zeros((NB, 256), jnp.float32)
    for k in range(K):
        e = jnp.exp(l3[k] - m)
        num = num + e * v3[k]
        den = den + e
    out_ref[...] = num / den


def _mlp_call(g, kfeat, kpos, pw1, pb1, pw2, pb2, aw1, ab1, aw2, ab2, b, n):
    nblk = n // NB
    grid = (b, nblk)
    gv = g.reshape(b * K, n, DROW)
    return pl.pallas_call(
        _mlp_body,
        grid=grid,
        in_specs=[
            pl.BlockSpec((K, NB, DROW), lambda i, j: (i, j, 0)),
            pl.BlockSpec((NB, 256), lambda i, j: (i * nblk + j, 0)),
            pl.BlockSpec((NB, DPAD), lambda i, j: (i * nblk + j, 0)),
            pl.BlockSpec((DPAD, 64), lambda i, j: (0, 0)),
            pl.BlockSpec((1, 64), lambda i, j: (0, 0)),
            pl.BlockSpec((64, 256), lambda i, j: (0, 0)),
            pl.BlockSpec((1, 256), lambda i, j: (0, 0)),
            pl.BlockSpec((256, 1024), lambda i, j: (0, 0)),
            pl.BlockSpec((1, 1024), lambda i, j: (0, 0)),
            pl.BlockSpec((1024, 256), lambda i, j: (0, 0)),
            pl.BlockSpec((1, 256), lambda i, j: (0, 0)),
        ],
        out_specs=pl.BlockSpec((NB, 256), lambda i, j: (i * nblk + j, 0)),
        out_shape=jax.ShapeDtypeStruct((b * n, 256), jnp.float32),
    )(gv, kfeat, kpos, pw1, pb1, pw2, pb2, aw1, ab1, aw2, ab2)


def kernel(pcd, feat, pcd_feadb, feat_feadb,
           pos_w1, pos_b1, pos_g1, pos_beta1, pos_w2, pos_b2,
           attn_w1, attn_b1, attn_g1, attn_beta1, attn_w2, attn_b2):
    b, _, n = pcd.shape
    nf = pcd_feadb.shape[2]
    nt = n + nf

    fusion_pcd = jnp.concatenate((pcd, pcd_feadb), axis=2)
    fusion_feat = jnp.concatenate((feat, feat_feadb), axis=2)

    qpos3 = jnp.pad(jnp.transpose(pcd, (0, 2, 1)), ((0, 0), (0, 0), (0, 5)))
    fpos = jnp.pad(fusion_pcd, ((0, 0), (0, 5), (0, 0)))

    feat_p = lax.bitcast_convert_type(
        jnp.transpose(fusion_feat, (0, 2, 1)).astype(jnp.bfloat16)
        .reshape(b * nt, 128, 2), jnp.float32)
    pos_t = jnp.pad(jnp.transpose(fusion_pcd, (0, 2, 1)),
                    ((0, 0), (0, 0), (0, DPAD - 3))).reshape(b * nt, DPAD)
    zpad = jnp.zeros((b * nt, DROW - 128 - DPAD), jnp.float32)
    table = jnp.concatenate((feat_p, pos_t, zpad), axis=1)

    ps = pos_g1 / jnp.sqrt(1.0 + EPS)
    pw1 = jnp.pad(pos_w1, ((0, 0), (0, DPAD - 3))).T * ps[None, :]
    pb1 = (pos_b1 * ps + pos_beta1)[None, :]
    pw2 = pos_w2.T
    pb2 = pos_b2[None, :]
    as_ = attn_g1 / jnp.sqrt(1.0 + EPS)
    aw1 = (attn_w1.T * as_[None, :]).astype(jnp.bfloat16)
    ab1 = (attn_b1 * as_ + attn_beta1)[None, :].astype(jnp.bfloat16)
    aw2 = attn_w2.T.astype(jnp.bfloat16)
    ab2 = attn_b2[None, :]

    aw1 = jnp.concatenate((aw1[0::2], aw1[1::2]), axis=0)
    pw2 = jnp.concatenate((pw2[:, 0::2], pw2[:, 1::2]), axis=1)
    pb2 = jnp.concatenate((pb2[:, 0::2], pb2[:, 1::2]), axis=1)
    aw2 = jnp.concatenate((aw2[:, 0::2], aw2[:, 1::2]), axis=1)
    ab2 = jnp.concatenate((ab2[:, 0::2], ab2[:, 1::2]), axis=1)

    kfeat3 = jnp.transpose(feat, (0, 2, 1))
    kfeat3 = jnp.concatenate((kfeat3[..., 0::2], kfeat3[..., 1::2]), axis=-1)
    kpos3 = jnp.pad(jnp.transpose(pcd, (0, 2, 1)),
                    ((0, 0), (0, 0), (0, DPAD - 3)))

    nch = n // NCHUNK
    boff = (jnp.arange(b, dtype=jnp.int32) * nt)[:, None, None]
    outs = []
    for c in range(NCHUNK):
        sl = slice(c * nch, (c + 1) * nch)
        qpos_c = qpos3[:, sl].reshape(b * nch, 8)
        idx = _knn_call(qpos_c, fpos, b, nch, nt)
        idx3 = idx.reshape(b, nch, K) + boff
        idx_flat = jnp.transpose(idx3, (0, 2, 1)).reshape(b * K * nch)
        g = _gather_call(table, idx_flat)
        kfeat_c = kfeat3[:, sl].reshape(b * nch, 256)
        kpos_c = kpos3[:, sl].reshape(b * nch, DPAD)
        out_c = _mlp_call(g, kfeat_c, kpos_c, pw1, pb1, pw2, pb2,
                          aw1, ab1, aw2, ab2, b, nch)
        outs.append(out_c.reshape(b, nch, 256))
    out = jnp.concatenate(outs, axis=1)
    out = jnp.stack((out[..., :128], out[..., 128:]), axis=-1).reshape(b, n, 256)
    return jnp.transpose(out, (0, 2, 1))


# speedup vs baseline: 1.1740x; 1.1740x over previous
import functools

import jax
import jax.numpy as jnp
from jax import lax
from jax.experimental import pallas as pl
from jax.experimental.pallas import tpu as pltpu
from jax.experimental.pallas import tpu_sc as plsc

K = 16
EPS = 1e-5
NA = 256
NB = 128
DPAD = 16
DROW = 256 + 128

SC_NC = 2
SC_NS = 16
SC_NW = SC_NC * SC_NS
GCHUNK = 256
NCHUNK = 2


def _ce(v, i, a, b):
    le = v[a] <= v[b]
    va, vb = jnp.where(le, v[a], v[b]), jnp.where(le, v[b], v[a])
    ia, ib = jnp.where(le, i[a], i[b]), jnp.where(le, i[b], i[a])
    v[a], v[b], i[a], i[b] = va, vb, ia, ib


def _knn_body(q_ref, p_ref, idx_ref):
    q = q_ref[...]
    p = p_ref[0]
    nt = p.shape[1]
    w = nt // K
    psq = jnp.sum(p * p, axis=0, keepdims=True)
    d = psq - 2.0 * jnp.dot(q, p, preferred_element_type=jnp.float32)
    iota = lax.broadcasted_iota(jnp.int32, (d.shape[0], w), 1)
    v = [d[:, c * w:(c + 1) * w] for c in range(K)]
    i = [iota + c * w for c in range(K)]

    for k in (2, 4, 8, 16):
        j = k // 2
        while j >= 1:
            for a in range(K):
                b = a ^ j
                if b > a:
                    if (a & k) == 0:
                        _ce(v, i, a, b)
                    else:
                        _ce(v, i, b, a)
            j //= 2

    while w > 32:
        h = w // 2
        lo_v = [x[:, :h] for x in v]
        hi_v = [x[:, h:] for x in v]
        lo_i = [x[:, :h] for x in i]
        hi_i = [x[:, h:] for x in i]
        for a in range(K):
            le = lo_v[a] <= hi_v[K - 1 - a]
            v[a] = jnp.where(le, lo_v[a], hi_v[K - 1 - a])
            i[a] = jnp.where(le, lo_i[a], hi_i[K - 1 - a])
        w = h
        if w > 32:
            for j in (8, 4, 2, 1):
                for a in range(K):
                    b = a ^ j
                    if b > a:
                        _ce(v, i, a, b)

    dv = jnp.concatenate(v, axis=1)
    di = jnp.concatenate(i, axis=1)
    col = jax.lax.broadcasted_iota(jnp.int32, (dv.shape[0], K), 1)
    acc = jnp.zeros((dv.shape[0], K), jnp.int32)
    big = jnp.float32(3e38)
    for k in range(K):
        rmin = jnp.min(dv, axis=1, keepdims=True)
        eqm = dv == rmin
        cand = jnp.where(eqm, di, jnp.int32(1 << 30))
        sel = jnp.min(cand, axis=1, keepdims=True)
        dv = jnp.where(eqm & (di == sel), big, dv)
        acc = jnp.where(col == k, sel, acc)
    idx_ref[...] = acc


def _knn_call(qpos, fpos, b, n, nt):
    grid = (b, n // NA)
    return pl.pallas_call(
        _knn_body,
        grid=grid,
        in_specs=[
            pl.BlockSpec((NA, 8), lambda i, j: (i * (n // NA) + j, 0)),
            pl.BlockSpec((1, 8, nt), lambda i, j: (i, 0, 0)),
        ],
        out_specs=pl.BlockSpec((NA, K), lambda i, j: (i * (n // NA) + j, 0)),
        out_shape=jax.ShapeDtypeStruct((b * n, K), jnp.int32),
    )(qpos, fpos)


def _gather_body(table_hbm, idx_hbm, out_hbm, idx_v, rows_v, sem):
    wid = lax.axis_index("s") * SC_NC + lax.axis_index("c")
    rows_total = out_hbm.shape[0]
    per_w = rows_total // SC_NW
    base = wid * per_w

    def body(j, carry):
        off = base + j * GCHUNK
        pltpu.sync_copy(idx_hbm.at[pl.ds(off, GCHUNK)], idx_v)
        pltpu.async_copy(table_hbm.at[idx_v], rows_v, sem).wait()
        pltpu.sync_copy(rows_v, out_hbm.at[pl.ds(off, GCHUNK)])
        return carry

    lax.fori_loop(0, per_w // GCHUNK, body, 0)


def _gather_call(table, idx_flat):
    rows = idx_flat.shape[0]
    mesh = plsc.VectorSubcoreMesh(core_axis_name="c", subcore_axis_name="s")
    f = functools.partial(
        pl.kernel,
        mesh=mesh,
        out_type=jax.ShapeDtypeStruct((rows, DROW), jnp.float32),
        scratch_types=[
            pltpu.VMEM((GCHUNK,), jnp.int32),
            pltpu.VMEM((GCHUNK, DROW), jnp.float32),
            pltpu.SemaphoreType.DMA,
        ],
    )(_gather_body)
    return f(table, idx_flat)


def _mlp_body(g_ref, kf_ref, kp_ref, pw1_ref, pb1_ref, pw2_ref, pb2_ref,
              aw1_ref, ab1_ref, aw2_ref, ab2_ref, out_ref):
    g = g_ref[...].reshape(K * NB, DROW)
    gfeat = g[:, :256]
    gpos = g[:, 256:256 + DPAD]
    kf = kf_ref[...]
    kp = kp_ref[...]
    kfb = (jnp.broadcast_to(kf[None], (K, NB, 256)) - gfeat.reshape(K, NB, 256)
           ).reshape(K * NB, 256)
    kpb = (jnp.broadcast_to(kp[None], (K, NB, DPAD)) - gpos.reshape(K, NB, DPAD)
           ).reshape(K * NB, DPAD)

    h = jnp.maximum(
        jnp.dot(kpb, pw1_ref[...], preferred_element_type=jnp.float32)
        + pb1_ref[...], 0.0)
    pe = jnp.dot(h, pw2_ref[...], preferred_element_type=jnp.float32) + pb2_ref[...]

    x = (kfb + pe).astype(jnp.bfloat16)
    a = jnp.maximum(
        jnp.dot(x, aw1_ref[...],
                preferred_element_type=jnp.float32).astype(jnp.bfloat16)
        + ab1_ref[...], jnp.bfloat16(0.0))
    logits = (jnp.dot(a, aw2_ref[...], preferred_element_type=jnp.float32)
              + ab2_ref[...])

    l3 = logits.reshape(K, NB, 256)
    v3 = (gfeat + pe).reshape(K, NB, 256)
    m = l3[0]
    for k in range(1, K):
        m = jnp.maximum(m, l3[k])
    num = jnp.zeros((NB, 256), jnp.float32)
    den = jnp.zeros((NB, 256), jnp.float32)
    for k in range(K):
        e = jnp.exp(l3[k] - m)
        num = num + e * v3[k]
        den = den + e
    out_ref[...] = num / den


def _mlp_call(g, kfeat, kpos, pw1, pb1, pw2, pb2, aw1, ab1, aw2, ab2, b, n):
    nblk = n // NB
    grid = (b, nblk)
    gv = g.reshape(b * K, n, DROW)
    return pl.pallas_call(
        _mlp_body,
        grid=grid,
        in_specs=[
            pl.BlockSpec((K, NB, DROW), lambda i, j: (i, j, 0)),
            pl.BlockSpec((NB, 256), lambda i, j: (i * nblk + j, 0)),
            pl.BlockSpec((NB, DPAD), lambda i, j: (i * nblk + j, 0)),
            pl.BlockSpec((DPAD, 64), lambda i, j: (0, 0)),
            pl.BlockSpec((1, 64), lambda i, j: (0, 0)),
            pl.BlockSpec((64, 256), lambda i, j: (0, 0)),
            pl.BlockSpec((1, 256), lambda i, j: (0, 0)),
            pl.BlockSpec((256, 1024), lambda i, j: (0, 0)),
            pl.BlockSpec((1, 1024), lambda i, j: (0, 0)),
            pl.BlockSpec((1024, 256), lambda i, j: (0, 0)),
            pl.BlockSpec((1, 256), lambda i, j: (0, 0)),
        ],
        out_specs=pl.BlockSpec((NB, 256), lambda i, j: (i * nblk + j, 0)),
        out_shape=jax.ShapeDtypeStruct((b * n, 256), jnp.float32),
    )(gv, kfeat, kpos, pw1, pb1, pw2, pb2, aw1, ab1, aw2, ab2)


def kernel(pcd, feat, pcd_feadb, feat_feadb,
           pos_w1, pos_b1, pos_g1, pos_beta1, pos_w2, pos_b2,
           attn_w1, attn_b1, attn_g1, attn_beta1, attn_w2, attn_b2):
    b, _, n = pcd.shape
    nf = pcd_feadb.shape[2]
    nt = n + nf

    fusion_pcd = jnp.concatenate((pcd, pcd_feadb), axis=2)
    fusion_feat = jnp.concatenate((feat, feat_feadb), axis=2)

    qpos3 = jnp.pad(jnp.transpose(pcd, (0, 2, 1)), ((0, 0), (0, 0), (0, 5)))
    fpos = jnp.pad(fusion_pcd, ((0, 0), (0, 5), (0, 0)))

    feat_t = jnp.transpose(fusion_feat, (0, 2, 1)).reshape(b * nt, 256)
    pos_t = jnp.pad(jnp.transpose(fusion_pcd, (0, 2, 1)),
                    ((0, 0), (0, 0), (0, 128 - 3))).reshape(b * nt, 128)
    table = jnp.concatenate((feat_t, pos_t), axis=1)

    ps = pos_g1 / jnp.sqrt(1.0 + EPS)
    pw1 = jnp.pad(pos_w1, ((0, 0), (0, DPAD - 3))).T * ps[None, :]
    pb1 = (pos_b1 * ps + pos_beta1)[None, :]
    pw2 = pos_w2.T
    pb2 = pos_b2[None, :]
    as_ = attn_g1 / jnp.sqrt(1.0 + EPS)
    aw1 = (attn_w1.T * as_[None, :]).astype(jnp.bfloat16)
    ab1 = (attn_b1 * as_ + attn_beta1)[None, :].astype(jnp.bfloat16)
    aw2 = attn_w2.T.astype(jnp.bfloat16)
    ab2 = attn_b2[None, :]

    kfeat3 = jnp.transpose(feat, (0, 2, 1))
    kpos3 = jnp.pad(jnp.transpose(pcd, (0, 2, 1)),
                    ((0, 0), (0, 0), (0, DPAD - 3)))

    nch = n // NCHUNK
    boff = (jnp.arange(b, dtype=jnp.int32) * nt)[:, None, None]
    outs = []
    for c in range(NCHUNK):
        sl = slice(c * nch, (c + 1) * nch)
        qpos_c = qpos3[:, sl].reshape(b * nch, 8)
        idx = _knn_call(qpos_c, fpos, b, nch, nt)
        idx3 = idx.reshape(b, nch, K) + boff
        idx_flat = jnp.transpose(idx3, (0, 2, 1)).reshape(b * K * nch)
        g = _gather_call(table, idx_flat)
        kfeat_c = kfeat3[:, sl].reshape(b * nch, 256)
        kpos_c = kpos3[:, sl].reshape(b * nch, DPAD)
        out_c = _mlp_call(g, kfeat_c, kpos_c, pw1, pb1, pw2, pb2,
                          aw1, ab1, aw2, ab2, b, nch)
        outs.append(out_c.reshape(b, nch, 256))
    out = jnp.concatenate(outs, axis=1)
    return jnp.transpose(out, (0, 2, 1))


# speedup vs baseline: 1.2071x; 1.0282x over previous
import functools

import jax
import jax.numpy as jnp
from jax import lax
from jax.experimental import pallas as pl
from jax.experimental.pallas import tpu as pltpu
from jax.experimental.pallas import tpu_sc as plsc

K = 16
EPS = 1e-5
NA = 256
NB = 256
DPAD = 16
DROW = 256 + 128

SC_NC = 2
SC_NS = 16
SC_NW = SC_NC * SC_NS
GCHUNK = 256
NCHUNK = 2


def _ce(v, i, a, b):
    le = v[a] <= v[b]
    va, vb = jnp.where(le, v[a], v[b]), jnp.where(le, v[b], v[a])
    ia, ib = jnp.where(le, i[a], i[b]), jnp.where(le, i[b], i[a])
    v[a], v[b], i[a], i[b] = va, vb, ia, ib


def _knn_body(q_ref, p_ref, idx_ref):
    q = q_ref[...]
    p = p_ref[0]
    nt = p.shape[1]
    w = nt // K
    psq = jnp.sum(p * p, axis=0, keepdims=True)
    d = psq - 2.0 * jnp.dot(q, p, preferred_element_type=jnp.float32)
    iota = lax.broadcasted_iota(jnp.int32, (d.shape[0], w), 1)
    v = [d[:, c * w:(c + 1) * w] for c in range(K)]
    i = [iota + c * w for c in range(K)]

    for k in (2, 4, 8, 16):
        j = k // 2
        while j >= 1:
            for a in range(K):
                b = a ^ j
                if b > a:
                    if (a & k) == 0:
                        _ce(v, i, a, b)
                    else:
                        _ce(v, i, b, a)
            j //= 2

    while w > 32:
        h = w // 2
        lo_v = [x[:, :h] for x in v]
        hi_v = [x[:, h:] for x in v]
        lo_i = [x[:, :h] for x in i]
        hi_i = [x[:, h:] for x in i]
        for a in range(K):
            le = lo_v[a] <= hi_v[K - 1 - a]
            v[a] = jnp.where(le, lo_v[a], hi_v[K - 1 - a])
            i[a] = jnp.where(le, lo_i[a], hi_i[K - 1 - a])
        w = h
        if w > 32:
            for j in (8, 4, 2, 1):
                for a in range(K):
                    b = a ^ j
                    if b > a:
                        _ce(v, i, a, b)

    dv = jnp.concatenate(v, axis=1)
    di = jnp.concatenate(i, axis=1)
    col = jax.lax.broadcasted_iota(jnp.int32, (dv.shape[0], K), 1)
    acc = jnp.zeros((dv.shape[0], K), jnp.int32)
    big = jnp.float32(3e38)
    for k in range(K):
        rmin = jnp.min(dv, axis=1, keepdims=True)
        eqm = dv == rmin
        cand = jnp.where(eqm, di, jnp.int32(1 << 30))
        sel = jnp.min(cand, axis=1, keepdims=True)
        dv = jnp.where(eqm & (di == sel), big, dv)
        acc = jnp.where(col == k, sel, acc)
    idx_ref[...] = acc


def _knn_call(qpos, fpos, b, n, nt):
    grid = (b, n // NA)
    return pl.pallas_call(
        _knn_body,
        grid=grid,
        in_specs=[
            pl.BlockSpec((NA, 8), lambda i, j: (i * (n // NA) + j, 0)),
            pl.BlockSpec((1, 8, nt), lambda i, j: (i, 0, 0)),
        ],
        out_specs=pl.BlockSpec((NA, K), lambda i, j: (i * (n // NA) + j, 0)),
        out_shape=jax.ShapeDtypeStruct((b * n, K), jnp.int32),
    )(qpos, fpos)


def _gather_body(table_hbm, idx_hbm, out_hbm, idx_v, rows_v, sem):
    wid = lax.axis_index("s") * SC_NC + lax.axis_index("c")
    rows_total = out_hbm.shape[0]
    per_w = rows_total // SC_NW
    base = wid * per_w

    def body(j, carry):
        off = base + j * GCHUNK
        pltpu.sync_copy(idx_hbm.at[pl.ds(off, GCHUNK)], idx_v)
        pltpu.async_copy(table_hbm.at[idx_v], rows_v, sem).wait()
        pltpu.sync_copy(rows_v, out_hbm.at[pl.ds(off, GCHUNK)])
        return carry

    lax.fori_loop(0, per_w // GCHUNK, body, 0)


def _gather_call(table, idx_flat):
    rows = idx_flat.shape[0]
    mesh = plsc.VectorSubcoreMesh(core_axis_name="c", subcore_axis_name="s")
    f = functools.partial(
        pl.kernel,
        mesh=mesh,
        out_type=jax.ShapeDtypeStruct((rows, DROW), jnp.float32),
        scratch_types=[
            pltpu.VMEM((GCHUNK,), jnp.int32),
            pltpu.VMEM((GCHUNK, DROW), jnp.float32),
            pltpu.SemaphoreType.DMA,
        ],
    )(_gather_body)
    return f(table, idx_flat)


def _mlp_body(g_ref, kf_ref, kp_ref, pw1_ref, pb1_ref, pw2_ref, pb2_ref,
              aw1_ref, ab1_ref, aw2_ref, ab2_ref, out_ref):
    g = g_ref[...].reshape(K * NB, DROW)
    gfeat = g[:, :256]
    gpos = g[:, 256:256 + DPAD]
    kf = kf_ref[...]
    kp = kp_ref[...]
    kfb = (jnp.broadcast_to(kf[None], (K, NB, 256)) - gfeat.reshape(K, NB, 256)
           ).reshape(K * NB, 256)
    kpb = (jnp.broadcast_to(kp[None], (K, NB, DPAD)) - gpos.reshape(K, NB, DPAD)
           ).reshape(K * NB, DPAD)

    h = jnp.maximum(
        jnp.dot(kpb, pw1_ref[...], preferred_element_type=jnp.float32)
        + pb1_ref[...], 0.0)
    pe = jnp.dot(h, pw2_ref[...], preferred_element_type=jnp.float32) + pb2_ref[...]

    x = (kfb + pe).astype(jnp.bfloat16)
    a = jnp.maximum(
        jnp.dot(x, aw1_ref[...],
                preferred_element_type=jnp.float32).astype(jnp.bfloat16)
        + ab1_ref[...], jnp.bfloat16(0.0))
    logits = (jnp.dot(a, aw2_ref[...], preferred_element_type=jnp.float32)
              + ab2_ref[...])

    l3 = logits.reshape(K, NB, 256)
    v3 = (gfeat + pe).reshape(K, NB, 256)
    m = l3[0]
    for k in range(1, K):
        m = jnp.maximum(m, l3[k])
    num = jnp.zeros((NB, 256), jnp.float32)
    den = jnp.zeros((NB, 256), jnp.float32)
    for k in range(K):
        e = jnp.exp(l3[k] - m)
        num = num + e * v3[k]
        den = den + e
    out_ref[...] = num / den


def _mlp_call(g, kfeat, kpos, pw1, pb1, pw2, pb2, aw1, ab1, aw2, ab2, b, n):
    nblk = n // NB
    grid = (b, nblk)
    gv = g.reshape(b * K, n, DROW)
    return pl.pallas_call(
        _mlp_body,
        grid=grid,
        in_specs=[
            pl.BlockSpec((K, NB, DROW), lambda i, j: (i, j, 0)),
            pl.BlockSpec((NB, 256), lambda i, j: (i * nblk + j, 0)),
            pl.BlockSpec((NB, DPAD), lambda i, j: (i * nblk + j, 0)),
            pl.BlockSpec((DPAD, 64), lambda i, j: (0, 0)),
            pl.BlockSpec((1, 64), lambda i, j: (0, 0)),
            pl.BlockSpec((64, 256), lambda i, j: (0, 0)),
            pl.BlockSpec((1, 256), lambda i, j: (0, 0)),
            pl.BlockSpec((256, 1024), lambda i, j: (0, 0)),
            pl.BlockSpec((1, 1024), lambda i, j: (0, 0)),
            pl.BlockSpec((1024, 256), lambda i, j: (0, 0)),
            pl.BlockSpec((1, 256), lambda i, j: (0, 0)),
        ],
        out_specs=pl.BlockSpec((NB, 256), lambda i, j: (i * nblk + j, 0)),
        out_shape=jax.ShapeDtypeStruct((b * n, 256), jnp.float32),
    )(gv, kfeat, kpos, pw1, pb1, pw2, pb2, aw1, ab1, aw2, ab2)


def kernel(pcd, feat, pcd_feadb, feat_feadb,
           pos_w1, pos_b1, pos_g1, pos_beta1, pos_w2, pos_b2,
           attn_w1, attn_b1, attn_g1, attn_beta1, attn_w2, attn_b2):
    b, _, n = pcd.shape
    nf = pcd_feadb.shape[2]
    nt = n + nf

    fusion_pcd = jnp.concatenate((pcd, pcd_feadb), axis=2)
    fusion_feat = jnp.concatenate((feat, feat_feadb), axis=2)

    qpos3 = jnp.pad(jnp.transpose(pcd, (0, 2, 1)), ((0, 0), (0, 0), (0, 5)))
    fpos = jnp.pad(fusion_pcd, ((0, 0), (0, 5), (0, 0)))

    feat_t = jnp.transpose(fusion_feat, (0, 2, 1)).reshape(b * nt, 256)
    pos_t = jnp.pad(jnp.transpose(fusion_pcd, (0, 2, 1)),
                    ((0, 0), (0, 0), (0, 128 - 3))).reshape(b * nt, 128)
    table = jnp.concatenate((feat_t, pos_t), axis=1)

    ps = pos_g1 / jnp.sqrt(1.0 + EPS)
    pw1 = jnp.pad(pos_w1, ((0, 0), (0, DPAD - 3))).T * ps[None, :]
    pb1 = (pos_b1 * ps + pos_beta1)[None, :]
    pw2 = pos_w2.T
    pb2 = pos_b2[None, :]
    as_ = attn_g1 / jnp.sqrt(1.0 + EPS)
    aw1 = (attn_w1.T * as_[None, :]).astype(jnp.bfloat16)
    ab1 = (attn_b1 * as_ + attn_beta1)[None, :].astype(jnp.bfloat16)
    aw2 = attn_w2.T.astype(jnp.bfloat16)
    ab2 = attn_b2[None, :]

    kfeat3 = jnp.transpose(feat, (0, 2, 1))
    kpos3 = jnp.pad(jnp.transpose(pcd, (0, 2, 1)),
                    ((0, 0), (0, 0), (0, DPAD - 3)))

    nch = n // NCHUNK
    boff = (jnp.arange(b, dtype=jnp.int32) * nt)[:, None, None]
    outs = []
    for c in range(NCHUNK):
        sl = slice(c * nch, (c + 1) * nch)
        qpos_c = qpos3[:, sl].reshape(b * nch, 8)
        idx = _knn_call(qpos_c, fpos, b, nch, nt)
        idx3 = idx.reshape(b, nch, K) + boff
        idx_flat = jnp.transpose(idx3, (0, 2, 1)).reshape(b * K * nch)
        g = _gather_call(table, idx_flat)
        kfeat_c = kfeat3[:, sl].reshape(b * nch, 256)
        kpos_c = kpos3[:, sl].reshape(b * nch, DPAD)
        out_c = _mlp_call(g, kfeat_c, kpos_c, pw1, pb1, pw2, pb2,
                          aw1, ab1, aw2, ab2, b, nch)
        outs.append(out_c.reshape(b, nch, 256))
    out = jnp.concatenate(outs, axis=1)
    return jnp.transpose(out, (0, 2, 1))
